# Initial kernel scaffold; baseline (speedup 1.0000x reference)
import jax
import jax.numpy as jnp
from jax.experimental import pallas as pl


def kernel(featuremap, boxes, box_ind):
    raise NotImplementedError("write your pallas kernel here")


# speedup vs baseline: 7.8130x; 7.8130x over previous
import functools

import jax
import jax.numpy as jnp
from jax import lax
from jax.experimental import pallas as pl
from jax.experimental.pallas import tpu as pltpu
from jax.experimental.pallas import tpu_sc as plsc

CROP = 14
NPIX = CROP * CROP
NC, NS, L = 2, 16, 16
NW = NC * NS

P = 32


def _sc_blend(tbl, idx, wgt, *, m_total, c):
    pix_per_w = m_total // NW
    n_chunks = pix_per_w // P
    cc16 = c // L

    @functools.partial(
        pl.kernel,
        out_type=jax.ShapeDtypeStruct((m_total, c), jnp.float32),
        mesh=plsc.VectorSubcoreMesh(core_axis_name="c", subcore_axis_name="s"),
        scratch_types=[
            pltpu.VMEM((4 * P,), jnp.int32),
            pltpu.VMEM((4 * P,), jnp.float32),
            pltpu.VMEM((4 * P, c), jnp.float32),
            pltpu.VMEM((P, c), jnp.float32),
            pltpu.SemaphoreType.DMA,
        ],
        compiler_params=pltpu.CompilerParams(needs_layout_passes=False),
    )
    def k(tbl_hbm, idx_hbm, wgt_hbm, out_hbm, idx_v, wgt_v, rows_v, out_v, sem):
        wid = lax.axis_index("s") * NC + lax.axis_index("c")

        def chunk_body(g, carry):
            pix0 = wid * pix_per_w + g * P
            pltpu.sync_copy(idx_hbm.at[pl.ds(pix0 * 4, 4 * P)], idx_v)
            pltpu.sync_copy(wgt_hbm.at[pl.ds(pix0 * 4, 4 * P)], wgt_v)
            pltpu.async_copy(tbl_hbm.at[idx_v], rows_v, sem).wait()

            def pix_body(p, carry2):
                w0 = plsc.load_gather(wgt_v, [jnp.full((L,), 4 * p, jnp.int32)])
                w1 = plsc.load_gather(wgt_v, [jnp.full((L,), 4 * p + 1, jnp.int32)])
                w2 = plsc.load_gather(wgt_v, [jnp.full((L,), 4 * p + 2, jnp.int32)])
                w3 = plsc.load_gather(wgt_v, [jnp.full((L,), 4 * p + 3, jnp.int32)])
                for cc in range(cc16):
                    o = cc * L
                    acc = w0 * rows_v[4 * p, pl.ds(o, L)]
                    acc += w1 * rows_v[4 * p + 1, pl.ds(o, L)]
                    acc += w2 * rows_v[4 * p + 2, pl.ds(o, L)]
                    acc += w3 * rows_v[4 * p + 3, pl.ds(o, L)]
                    out_v[p, pl.ds(o, L)] = acc
                return carry2

            lax.fori_loop(0, P, pix_body, 0, unroll=False)
            pltpu.sync_copy(out_v, out_hbm.at[pl.ds(pix0, P)])
            return carry

        lax.fori_loop(0, n_chunks, chunk_body, 0, unroll=False)

    return k(tbl, idx, wgt)


def kernel(featuremap, boxes, box_ind):
    n, c, h, w = featuremap.shape
    m = boxes.shape[0]

    tbl = jnp.transpose(featuremap, (0, 2, 3, 1)).reshape(n * h * w, c)

    x1, y1, x2, y2 = boxes[:, 0], boxes[:, 1], boxes[:, 2], boxes[:, 3]
    spacing_w = (x2 - x1) / CROP
    spacing_h = (y2 - y1) / CROP
    nx0 = (x1 + spacing_w / 2 - 0.5) / (w - 1)
    ny0 = (y1 + spacing_h / 2 - 0.5) / (h - 1)
    nw_ = spacing_w * (CROP - 1) / (w - 1)
    nh_ = spacing_h * (CROP - 1) / (h - 1)
    g = jnp.linspace(0.0, 1.0, CROP)
    gy2, gx2 = jnp.meshgrid(g, g, indexing="ij")
    iy = (ny0[:, None, None] + nh_[:, None, None] * gy2[None]) * (h - 1)
    ix = (nx0[:, None, None] + nw_[:, None, None] * gx2[None]) * (w - 1)
    iy0 = jnp.floor(iy)
    ix0 = jnp.floor(ix)
    wy1 = iy - iy0
    wy0 = 1.0 - wy1
    wx1 = ix - ix0
    wx0 = 1.0 - wx1
    b = box_ind.astype(jnp.int32)[:, None, None]

    def corner(yc, xc, wgt):
        valid = (yc >= 0) & (yc <= h - 1) & (xc >= 0) & (xc <= w - 1)
        yci = jnp.clip(yc, 0, h - 1).astype(jnp.int32)
        xci = jnp.clip(xc, 0, w - 1).astype(jnp.int32)
        lin = (b * h + yci) * w + xci
        return lin, wgt * valid.astype(jnp.float32)

    i00, w00 = corner(iy0, ix0, wy0 * wx0)
    i01, w01 = corner(iy0, ix0 + 1.0, wy0 * wx1)
    i10, w10 = corner(iy0 + 1.0, ix0, wy1 * wx0)
    i11, w11 = corner(iy0 + 1.0, ix0 + 1.0, wy1 * wx1)
    idx = jnp.stack([i00, i01, i10, i11], axis=-1).reshape(-1)
    wgt = jnp.stack([w00, w01, w10, w11], axis=-1).reshape(-1)

    out = _sc_blend(tbl, idx, wgt, m_total=m * NPIX, c=c)
    return jnp.transpose(out.reshape(m, NPIX, c), (0, 2, 1)).reshape(m, c, CROP, CROP)
